# Initial kernel scaffold; baseline (speedup 1.0000x reference)
import jax
import jax.numpy as jnp
from jax.experimental import pallas as pl


def kernel(x, pos_table):
    raise NotImplementedError("write your pallas kernel here")


# speedup vs baseline: 1.4439x; 1.4439x over previous
import jax
import jax.numpy as jnp
from jax.experimental import pallas as pl

_SEQ_BLOCK = 512


def _add_body(x_ref, pos_ref, out_ref):
    out_ref[...] = x_ref[...] + pos_ref[...]


def kernel(x, pos_table):
    batch, seqlen, dim = x.shape
    n_seq = seqlen // _SEQ_BLOCK
    return pl.pallas_call(
        _add_body,
        grid=(n_seq, batch),
        in_specs=[
            pl.BlockSpec((1, _SEQ_BLOCK, dim), lambda s, b: (b, s, 0)),
            pl.BlockSpec((_SEQ_BLOCK, dim), lambda s, b: (s, 0)),
        ],
        out_specs=pl.BlockSpec((1, _SEQ_BLOCK, dim), lambda s, b: (b, s, 0)),
        out_shape=jax.ShapeDtypeStruct(x.shape, x.dtype),
    )(x, pos_table)
